# baseline (device time: 27000 ns/iter reference)
import jax
import jax.numpy as jnp
from jax import lax
from jax.experimental import pallas as pl
from jax.experimental.pallas import tpu as pltpu

N_LAYERS = 3


def kernel(x, Win0, Wout0, Win1, Wout1, Win2, Wout2):
    b, d_local = x.shape
    _, h_local = Win0.shape

    def body(x_ref, win0_ref, wout0_ref, win1_ref, wout1_ref, win2_ref,
             wout2_ref, out_ref,
             ysend_ref, yrecv_ref, xsend_ref, xrecv_ref,
             send_sems, recv_sems):
        my_x = lax.axis_index("x")
        my_y = lax.axis_index("y")
        y_nbr = (my_x, 1 - my_y)
        x_nbr = (1 - my_x, my_y)

        barrier_sem = pltpu.get_barrier_semaphore()
        for nbr in (y_nbr, x_nbr):
            pl.semaphore_signal(
                barrier_sem, inc=1,
                device_id=nbr, device_id_type=pl.DeviceIdType.MESH,
            )
        pl.semaphore_wait(barrier_sem, 2)

        wins = (win0_ref, win1_ref, win2_ref)
        wouts = (wout0_ref, wout1_ref, wout2_ref)

        x_cur = x_ref[...]
        for k in range(N_LAYERS):
            ysend_ref[...] = lax.dot_general(
                x_cur, wins[k][...],
                (((1,), (0,)), ((), ())),
                preferred_element_type=jnp.float32,
            )
            rdma_y = pltpu.make_async_remote_copy(
                src_ref=ysend_ref,
                dst_ref=yrecv_ref.at[k],
                send_sem=send_sems.at[2 * k],
                recv_sem=recv_sems.at[2 * k],
                device_id=y_nbr,
                device_id_type=pl.DeviceIdType.MESH,
            )
            rdma_y.start()
            rdma_y.wait()
            h = jnp.maximum(ysend_ref[...] + yrecv_ref[k], 0.0)

            xsend_ref[...] = lax.dot_general(
                h, wouts[k][...],
                (((1,), (0,)), ((), ())),
                preferred_element_type=jnp.float32,
            )
            rdma_x = pltpu.make_async_remote_copy(
                src_ref=xsend_ref,
                dst_ref=xrecv_ref.at[k],
                send_sem=send_sems.at[2 * k + 1],
                recv_sem=recv_sems.at[2 * k + 1],
                device_id=x_nbr,
                device_id_type=pl.DeviceIdType.MESH,
            )
            rdma_x.start()
            rdma_x.wait()
            x_cur = xsend_ref[...] + xrecv_ref[k]

        out_ref[...] = x_cur

    return pl.pallas_call(
        body,
        out_shape=jax.ShapeDtypeStruct((b, d_local), jnp.float32),
        in_specs=[pl.BlockSpec(memory_space=pltpu.VMEM)] * 7,
        out_specs=pl.BlockSpec(memory_space=pltpu.VMEM),
        scratch_shapes=[
            pltpu.VMEM((b, h_local), jnp.float32),
            pltpu.VMEM((N_LAYERS, b, h_local), jnp.float32),
            pltpu.VMEM((b, d_local), jnp.float32),
            pltpu.VMEM((N_LAYERS, b, d_local), jnp.float32),
            pltpu.SemaphoreType.DMA((2 * N_LAYERS,)),
            pltpu.SemaphoreType.DMA((2 * N_LAYERS,)),
        ],
        compiler_params=pltpu.CompilerParams(collective_id=0),
    )(x, Win0, Wout0, Win1, Wout1, Win2, Wout2)
